# baseline (speedup 1.0000x reference)
import functools

import jax
import jax.numpy as jnp
from jax import lax
from jax.experimental import pallas as pl
from jax.experimental.pallas import tpu as pltpu
from jax.experimental.pallas import tpu_sc as plsc

N, E, NG = 10000, 320000, 128
DX, DE = 128, 16
H, DOUT = 76, 64
HP = 128
NP = 10240
NC, NS, NW = 2, 16, 32
CH = 128
EPW = 10240
EP = NW * EPW
NCHUNK = EPW // CH

BN = 1024
BE = 4096

_SC_MESH = plsc.VectorSubcoreMesh(core_axis_name="c", subcore_axis_name="s")


def _node_init_body(x_ref, wex_ref, bex_ref, wni_ref, bni_ref, h_ref):
    hx = jax.nn.relu(jnp.dot(x_ref[...], wex_ref[...],
                             preferred_element_type=jnp.float32) + bex_ref[...])
    h_ref[...] = jax.nn.relu(jnp.dot(hx, wni_ref[...],
                                     preferred_element_type=jnp.float32) + bni_ref[...])


def _node_init(xp, Wex, bex, Wni_p, bni_p):
    grid = NP // BN
    return pl.pallas_call(
        _node_init_body,
        grid=(grid,),
        in_specs=[
            pl.BlockSpec((BN, DX), lambda i: (i, 0)),
            pl.BlockSpec((DX, DX), lambda i: (0, 0)),
            pl.BlockSpec((1, DX), lambda i: (0, 0)),
            pl.BlockSpec((DX, HP), lambda i: (0, 0)),
            pl.BlockSpec((1, HP), lambda i: (0, 0)),
        ],
        out_specs=pl.BlockSpec((BN, HP), lambda i: (i, 0)),
        out_shape=jax.ShapeDtypeStruct((NP, HP), jnp.float32),
    )(xp, Wex, bex, Wni_p, bni_p)


def _edge_init_body(ea_ref, wee_ref, bee_ref, wei_ref, bei_ref, e_ref):
    he = jax.nn.relu(jnp.dot(ea_ref[...], wee_ref[...],
                             preferred_element_type=jnp.float32) + bee_ref[...])
    e_ref[...] = jax.nn.relu(jnp.dot(he, wei_ref[...],
                                     preferred_element_type=jnp.float32) + bei_ref[...])


def _edge_init(eap, Wee, bee, Wei_p, bei_p):
    grid = EP // BE
    return pl.pallas_call(
        _edge_init_body,
        grid=(grid,),
        in_specs=[
            pl.BlockSpec((BE, DE), lambda i: (i, 0)),
            pl.BlockSpec((DE, DX), lambda i: (0, 0)),
            pl.BlockSpec((1, DX), lambda i: (0, 0)),
            pl.BlockSpec((DX, HP), lambda i: (0, 0)),
            pl.BlockSpec((1, HP), lambda i: (0, 0)),
        ],
        out_specs=pl.BlockSpec((BE, HP), lambda i: (i, 0)),
        out_shape=jax.ShapeDtypeStruct((EP, HP), jnp.float32),
    )(eap, Wee, bee, Wei_p, bei_p)


def _edge_update_body(e_ref, a_ref, b_ref, w3_ref, o_ref, *, ones_col):
    acc = jnp.dot(e_ref[...], w3_ref[...], preferred_element_type=jnp.float32)
    out = jax.nn.relu(acc + a_ref[...] + b_ref[...])
    if ones_col:
        col = lax.broadcasted_iota(jnp.int32, out.shape, 1)
        out = jnp.where(col == H, 1.0, out)
    o_ref[...] = out


def _edge_update(e, A, B, W3, ones_col):
    grid = EP // BE
    return pl.pallas_call(
        functools.partial(_edge_update_body, ones_col=ones_col),
        grid=(grid,),
        in_specs=[
            pl.BlockSpec((BE, HP), lambda i: (i, 0)),
            pl.BlockSpec((BE, HP), lambda i: (i, 0)),
            pl.BlockSpec((BE, HP), lambda i: (i, 0)),
            pl.BlockSpec((HP, HP), lambda i: (0, 0)),
        ],
        out_specs=pl.BlockSpec((BE, HP), lambda i: (i, 0)),
        out_shape=jax.ShapeDtypeStruct((EP, HP), jnp.float32),
    )(e, A, B, W3)


def _node_up_body(h_ref, m0_ref, m1_ref, wna_ref, wnb_ref, bn_ref,
                  wea_ref, bep_ref, web_ref, hn_ref, p1_ref, p2_ref):
    m = m0_ref[...] + m1_ref[...]
    hn = jax.nn.relu(
        jnp.dot(h_ref[...], wna_ref[...], preferred_element_type=jnp.float32)
        + jnp.dot(m, wnb_ref[...], preferred_element_type=jnp.float32)
        + bn_ref[...])
    hn_ref[...] = hn
    p1_ref[...] = jnp.dot(hn, wea_ref[...],
                          preferred_element_type=jnp.float32) + bep_ref[...]
    p2_ref[...] = jnp.dot(hn, web_ref[...], preferred_element_type=jnp.float32)


def _node_update_pp(h, m0, m1, WnA, WnB, bn_p, WeA, beP, WeB):
    grid = NP // BN
    blk = pl.BlockSpec((BN, HP), lambda i: (i, 0))
    wblk = pl.BlockSpec((HP, HP), lambda i: (0, 0))
    bblk = pl.BlockSpec((1, HP), lambda i: (0, 0))
    return pl.pallas_call(
        _node_up_body,
        grid=(grid,),
        in_specs=[blk, blk, blk, wblk, wblk, bblk, wblk, bblk, wblk],
        out_specs=[blk, blk, blk],
        out_shape=[jax.ShapeDtypeStruct((NP, HP), jnp.float32)] * 3,
    )(h, m0, m1, WnA, WnB, bn_p, WeA, beP, WeB)


def _pp_body(h_ref, wea_ref, bep_ref, web_ref, p1_ref, p2_ref):
    h = h_ref[...]
    p1_ref[...] = jnp.dot(h, wea_ref[...],
                          preferred_element_type=jnp.float32) + bep_ref[...]
    p2_ref[...] = jnp.dot(h, web_ref[...], preferred_element_type=jnp.float32)


def _pp(h, WeA, beP, WeB):
    grid = NP // BN
    return pl.pallas_call(
        _pp_body,
        grid=(grid,),
        in_specs=[
            pl.BlockSpec((BN, HP), lambda i: (i, 0)),
            pl.BlockSpec((HP, HP), lambda i: (0, 0)),
            pl.BlockSpec((1, HP), lambda i: (0, 0)),
            pl.BlockSpec((HP, HP), lambda i: (0, 0)),
        ],
        out_specs=[pl.BlockSpec((BN, HP), lambda i: (i, 0)),
                   pl.BlockSpec((BN, HP), lambda i: (i, 0))],
        out_shape=[jax.ShapeDtypeStruct((NP, HP), jnp.float32)] * 2,
    )(h, WeA, beP, WeB)


def _readout_body(s0_ref, s1_ref, batch_ref, weo_ref, wr1_ref, br1_ref,
                  wr2_ref, br2_ref, o_ref):
    s = s0_ref[...] + s1_ref[...]
    gids = lax.broadcasted_iota(jnp.int32, (NG, NP), 0)
    onehot = (gids == batch_ref[...]).astype(jnp.float32)
    pooled_aug = jnp.dot(onehot, s, preferred_element_type=jnp.float32)
    pooled = jnp.dot(pooled_aug, weo_ref[...], preferred_element_type=jnp.float32)
    r1 = jax.nn.relu(jnp.dot(pooled, wr1_ref[...],
                             preferred_element_type=jnp.float32) + br1_ref[...])
    o_ref[...] = jnp.dot(r1, wr2_ref[...],
                         preferred_element_type=jnp.float32) + br2_ref[...]


def _readout(s0, s1, batch_r, Weo_aug, Wr1, br1, Wr2, br2):
    return pl.pallas_call(
        _readout_body,
        in_specs=[
            pl.BlockSpec((NP, HP), lambda: (0, 0)),
            pl.BlockSpec((NP, HP), lambda: (0, 0)),
            pl.BlockSpec((1, NP), lambda: (0, 0)),
            pl.BlockSpec((HP, DOUT), lambda: (0, 0)),
            pl.BlockSpec((DOUT, 64), lambda: (0, 0)),
            pl.BlockSpec((1, 64), lambda: (0, 0)),
            pl.BlockSpec((64, 1), lambda: (0, 0)),
            pl.BlockSpec((1, 1), lambda: (0, 0)),
        ],
        out_specs=pl.BlockSpec((NG, 1), lambda: (0, 0)),
        out_shape=jax.ShapeDtypeStruct((NG, 1), jnp.float32),
    )(s0, s1, batch_r, Weo_aug, Wr1, br1, Wr2, br2)


SB = 2
TOTCH = EP // CH
CF, CS = 120, 40
IDXPAD = 16 * CF + 16 * CS + (CF - CS)
IDXPAD = ((IDXPAD + 7) // 8) * 8


def _sc_gather2(P1, P2, src_f, dst_f):
    @functools.partial(
        pl.kernel,
        mesh=_SC_MESH,
        out_type=(jax.ShapeDtypeStruct((EP, HP), jnp.float32),
                  jax.ShapeDtypeStruct((EP, HP), jnp.float32)),
        scratch_types=[
            pltpu.VMEM((CF, CH), jnp.int32),
            pltpu.VMEM((CF, CH), jnp.int32),
            pltpu.VMEM((SB * CH, HP), jnp.float32),
            pltpu.VMEM((SB * CH, HP), jnp.float32),
            pltpu.SemaphoreType.DMA,
            pltpu.SemaphoreType.DMA,
        ],
    )
    def k(p1_hbm, p2_hbm, src_hbm, dst_hbm, a_hbm, b_hbm,
          si_v, di_v, stag_a, stag_b, sema, semb):
        cid = lax.axis_index("c")
        sid = lax.axis_index("s")
        nsb = jnp.where(cid == 0, CF // SB, CS // SB)
        chunk0 = jnp.where(cid == 0, sid * CF, NS * CF + sid * CS)
        pltpu.sync_copy(src_hbm.at[pl.ds(chunk0, CF)], si_v)
        pltpu.sync_copy(dst_hbm.at[pl.ds(chunk0, CF)], di_v)

        def fire(table, idxv, k_sb, stag, sem):
            for j in range(SB):
                pltpu.async_copy(table.at[idxv.at[k_sb * SB + j]],
                                 stag.at[pl.ds(j * CH, CH)], sem)

        def drain(stag, sem):
            pltpu.make_async_copy(a_hbm.at[pl.ds(0, SB * CH)], stag, sem).wait()

        def out_slice(k_sb):
            return pl.ds((chunk0 + k_sb * SB) * CH, SB * CH)

        fire(p1_hbm, si_v, 0, stag_a, sema)

        @pl.loop(0, CF // SB)
        def _(k_sb):
            @pl.when(k_sb < nsb)
            def _():
                fire(p2_hbm, di_v, k_sb, stag_b, semb)
                drain(stag_a, sema)
                pltpu.sync_copy(stag_a, a_hbm.at[out_slice(k_sb)])

                @pl.when(k_sb + 1 < nsb)
                def _():
                    fire(p1_hbm, si_v, k_sb + 1, stag_a, sema)

                drain(stag_b, semb)
                pltpu.sync_copy(stag_b, b_hbm.at[out_slice(k_sb)])

    return k(P1, P2, src_f, dst_f)


def _sc_scatter(e, idx_r, zeros_np):
    @functools.partial(
        pl.kernel,
        mesh=_SC_MESH,
        out_type=jax.ShapeDtypeStruct((NC, NP, HP), jnp.float32),
        scratch_types=[
            pltpu.VMEM((NCHUNK, CH), jnp.int32),
            pltpu.VMEM((CH, HP), jnp.float32),
            pltpu.VMEM((CH, HP), jnp.float32),
            pltpu.VMEM_SHARED((NP, HP), jnp.float32),
            pltpu.SemaphoreType.DMA,
            pltpu.SemaphoreType.DMA,
        ],
    )
    def k(e_hbm, idx_hbm, z_hbm, out_hbm, idx_v, buf0, buf1, acc_sh, sem0, sem1):
        cid = lax.axis_index("c")
        sid = lax.axis_index("s")
        wid = cid * NS + sid
        base = wid * EPW
        rows = NP // NS
        pltpu.sync_copy(z_hbm.at[pl.ds(sid * rows, rows)],
                        acc_sh.at[pl.ds(sid * rows, rows)])
        plsc.subcore_barrier()
        pltpu.sync_copy(idx_hbm.at[wid], idx_v)

        bufs, sems = (buf0, buf1), (sem0, sem1)

        def lstart(ci, b):
            pltpu.async_copy(e_hbm.at[pl.ds(base + ci * CH, CH)], bufs[b], sems[b])

        lstart(0, 0)

        @pl.loop(0, NCHUNK, step=2)
        def _(c):
            for b in range(2):
                cc = c + b

                @pl.when(cc + 1 < NCHUNK)
                def _():
                    lstart(cc + 1, 1 - b)

                pltpu.make_async_copy(e_hbm.at[pl.ds(0, CH)],
                                      bufs[b], sems[b]).wait()
                pltpu.sync_copy(bufs[b], acc_sh.at[idx_v.at[cc]], add=True)

        plsc.subcore_barrier()

        @pl.when(sid == 0)
        def _():
            pltpu.sync_copy(acc_sh, out_hbm.at[cid])

    return k(e, idx_r, zeros_np)


def kernel(x, edge_attr, edge_index, batch, Wex, bex, Wee, bee, Wni, bni,
           Wei, bei, We, be, Wn, bn, Weo, beo, Wr1, br1, Wr2, br2):
    f32 = jnp.float32
    pad_w = lambda w: jnp.pad(w, ((0, HP - w.shape[0]), (0, HP - w.shape[1])))
    pad_c = lambda w: jnp.pad(w, ((0, 0), (0, HP - w.shape[1])))
    pad_b = lambda b: jnp.pad(b, (0, HP - b.shape[0])).reshape(1, HP)

    xp = jnp.pad(x, ((0, NP - N), (0, 0)))
    eap = jnp.pad(edge_attr, ((0, EP - E), (0, 0)))
    src_p = jnp.pad(edge_index[0], (0, EP - E), constant_values=N)
    dst_p = jnp.pad(edge_index[1], (0, EP - E), constant_values=N)
    flat = lambda v: jnp.pad(v.reshape(TOTCH, CH), ((0, IDXPAD - TOTCH), (0, 0)),
                             constant_values=N)
    src_f, dst_f = flat(src_p), flat(dst_p)
    src_r = src_p.reshape(NW, NCHUNK, CH)
    dst_r = dst_p.reshape(NW, NCHUNK, CH)
    batch_r = jnp.pad(batch, (0, NP - N), constant_values=NG).reshape(1, NP)

    Wni_p, bni_p = pad_c(Wni), pad_b(bni)
    Wei_p, bei_p = pad_c(Wei), pad_b(bei)
    WeA = [pad_w(We[l][:H]) for l in range(3)]
    WeB = [pad_w(We[l][H:2 * H]) for l in range(3)]
    W3 = [pad_w(We[l][2 * H:]) for l in range(3)]
    beP = [pad_b(be[l]) for l in range(3)]
    WnA = [pad_w(Wn[l][:H]) for l in range(2)]
    WnB = [pad_w(Wn[l][H:]) for l in range(2)]
    bn_p = [pad_b(bn[l]) for l in range(2)]
    Weo_aug = jnp.concatenate(
        [Weo, beo[None, :], jnp.zeros((HP - H - 1, DOUT), f32)], axis=0)
    zeros_np = jnp.zeros((NP, HP), f32)

    h = _node_init(xp, Wex, bex.reshape(1, DX), Wni_p, bni_p)
    e = _edge_init(eap, Wee, bee.reshape(1, DX), Wei_p, bei_p)
    P1, P2 = _pp(h, WeA[0], beP[0], WeB[0])

    for l in range(3):
        A, B = _sc_gather2(P1, P2, src_f, dst_f)
        e = _edge_update(e, A, B, W3[l], ones_col=(l == 2))
        parts = _sc_scatter(e, dst_r if l < 2 else src_r, zeros_np)
        if l < 2:
            h, P1, P2 = _node_update_pp(h, parts[0], parts[1], WnA[l], WnB[l],
                                        bn_p[l], WeA[l + 1], beP[l + 1], WeB[l + 1])

    return _readout(parts[0], parts[1], batch_r, Weo_aug,
                    Wr1, br1.reshape(1, 64), Wr2, br2.reshape(1, 1))


# speedup vs baseline: 1.0797x; 1.0797x over previous
import functools

import jax
import jax.numpy as jnp
from jax import lax
from jax.experimental import pallas as pl
from jax.experimental.pallas import tpu as pltpu
from jax.experimental.pallas import tpu_sc as plsc

N, E, NG = 10000, 320000, 128
DX, DE = 128, 16
H, DOUT = 76, 64
HP = 128
NP = 10240
NC, NS, NW = 2, 16, 32
CH = 128
EPW = 10240
EP = NW * EPW
NCHUNK = EPW // CH

BN = 1024
BE = 4096

_SC_MESH = plsc.VectorSubcoreMesh(core_axis_name="c", subcore_axis_name="s")


def _node_init_body(x_ref, wex_ref, bex_ref, wni_ref, bni_ref, h_ref):
    hx = jax.nn.relu(jnp.dot(x_ref[...], wex_ref[...],
                             preferred_element_type=jnp.float32) + bex_ref[...])
    h_ref[...] = jax.nn.relu(jnp.dot(hx, wni_ref[...],
                                     preferred_element_type=jnp.float32) + bni_ref[...])


def _node_init(xp, Wex, bex, Wni_p, bni_p):
    grid = NP // BN
    return pl.pallas_call(
        _node_init_body,
        grid=(grid,),
        in_specs=[
            pl.BlockSpec((BN, DX), lambda i: (i, 0)),
            pl.BlockSpec((DX, DX), lambda i: (0, 0)),
            pl.BlockSpec((1, DX), lambda i: (0, 0)),
            pl.BlockSpec((DX, HP), lambda i: (0, 0)),
            pl.BlockSpec((1, HP), lambda i: (0, 0)),
        ],
        out_specs=pl.BlockSpec((BN, HP), lambda i: (i, 0)),
        out_shape=jax.ShapeDtypeStruct((NP, HP), jnp.float32),
    )(xp, Wex, bex, Wni_p, bni_p)


def _edge_init_body(ea_ref, wee_ref, bee_ref, wei_ref, bei_ref, e_ref):
    he = jax.nn.relu(jnp.dot(ea_ref[...], wee_ref[...],
                             preferred_element_type=jnp.float32) + bee_ref[...])
    e_ref[...] = jax.nn.relu(jnp.dot(he, wei_ref[...],
                                     preferred_element_type=jnp.float32) + bei_ref[...])


def _edge_init(eap, Wee, bee, Wei_p, bei_p):
    grid = EP // BE
    return pl.pallas_call(
        _edge_init_body,
        grid=(grid,),
        in_specs=[
            pl.BlockSpec((BE, DE), lambda i: (i, 0)),
            pl.BlockSpec((DE, DX), lambda i: (0, 0)),
            pl.BlockSpec((1, DX), lambda i: (0, 0)),
            pl.BlockSpec((DX, HP), lambda i: (0, 0)),
            pl.BlockSpec((1, HP), lambda i: (0, 0)),
        ],
        out_specs=pl.BlockSpec((BE, HP), lambda i: (i, 0)),
        out_shape=jax.ShapeDtypeStruct((EP, HP), jnp.float32),
    )(eap, Wee, bee, Wei_p, bei_p)


def _edge_update_body(e_ref, a_ref, b_ref, w3_ref, o_ref):
    acc = jnp.dot(e_ref[...], w3_ref[...], preferred_element_type=jnp.float32)
    o_ref[...] = jax.nn.relu(acc + a_ref[...] + b_ref[...])


def _edge_out_body(e_ref, a_ref, b_ref, w3_ref, weo_ref, o_ref):
    acc = jnp.dot(e_ref[...], w3_ref[...], preferred_element_type=jnp.float32)
    out = jax.nn.relu(acc + a_ref[...] + b_ref[...])
    col = lax.broadcasted_iota(jnp.int32, out.shape, 1)
    out = jnp.where(col == H, 1.0, out)
    o_ref[...] = jnp.dot(out, weo_ref[...], preferred_element_type=jnp.float32)


def _edge_update(e, A, B, W3):
    grid = EP // BE
    return pl.pallas_call(
        _edge_update_body,
        grid=(grid,),
        in_specs=[
            pl.BlockSpec((BE, HP), lambda i: (i, 0)),
            pl.BlockSpec((BE, HP), lambda i: (i, 0)),
            pl.BlockSpec((BE, HP), lambda i: (i, 0)),
            pl.BlockSpec((HP, HP), lambda i: (0, 0)),
        ],
        out_specs=pl.BlockSpec((BE, HP), lambda i: (i, 0)),
        out_shape=jax.ShapeDtypeStruct((EP, HP), jnp.float32),
    )(e, A, B, W3)


def _edge_out(e, A, B, W3, Weo_aug):
    grid = EP // BE
    return pl.pallas_call(
        _edge_out_body,
        grid=(grid,),
        in_specs=[
            pl.BlockSpec((BE, HP), lambda i: (i, 0)),
            pl.BlockSpec((BE, HP), lambda i: (i, 0)),
            pl.BlockSpec((BE, HP), lambda i: (i, 0)),
            pl.BlockSpec((HP, HP), lambda i: (0, 0)),
            pl.BlockSpec((HP, HP), lambda i: (0, 0)),
        ],
        out_specs=pl.BlockSpec((BE, HP), lambda i: (i, 0)),
        out_shape=jax.ShapeDtypeStruct((EP, HP), jnp.float32),
    )(e, A, B, W3, Weo_aug)


def _node_up_body(h_ref, m0_ref, m1_ref, wna_ref, wnb_ref, bn_ref,
                  wea_ref, bep_ref, web_ref, hn_ref, p1_ref, p2_ref):
    m = m0_ref[...] + m1_ref[...]
    hn = jax.nn.relu(
        jnp.dot(h_ref[...], wna_ref[...], preferred_element_type=jnp.float32)
        + jnp.dot(m, wnb_ref[...], preferred_element_type=jnp.float32)
        + bn_ref[...])
    hn_ref[...] = hn
    p1_ref[...] = jnp.dot(hn, wea_ref[...],
                          preferred_element_type=jnp.float32) + bep_ref[...]
    p2_ref[...] = jnp.dot(hn, web_ref[...], preferred_element_type=jnp.float32)


def _node_update_pp(h, m0, m1, WnA, WnB, bn_p, WeA, beP, WeB):
    grid = NP // BN
    blk = pl.BlockSpec((BN, HP), lambda i: (i, 0))
    wblk = pl.BlockSpec((HP, HP), lambda i: (0, 0))
    bblk = pl.BlockSpec((1, HP), lambda i: (0, 0))
    return pl.pallas_call(
        _node_up_body,
        grid=(grid,),
        in_specs=[blk, blk, blk, wblk, wblk, bblk, wblk, bblk, wblk],
        out_specs=[blk, blk, blk],
        out_shape=[jax.ShapeDtypeStruct((NP, HP), jnp.float32)] * 3,
    )(h, m0, m1, WnA, WnB, bn_p, WeA, beP, WeB)


def _pp_body(h_ref, wea_ref, bep_ref, web_ref, p1_ref, p2_ref):
    h = h_ref[...]
    p1_ref[...] = jnp.dot(h, wea_ref[...],
                          preferred_element_type=jnp.float32) + bep_ref[...]
    p2_ref[...] = jnp.dot(h, web_ref[...], preferred_element_type=jnp.float32)


def _pp(h, WeA, beP, WeB):
    grid = NP // BN
    return pl.pallas_call(
        _pp_body,
        grid=(grid,),
        in_specs=[
            pl.BlockSpec((BN, HP), lambda i: (i, 0)),
            pl.BlockSpec((HP, HP), lambda i: (0, 0)),
            pl.BlockSpec((1, HP), lambda i: (0, 0)),
            pl.BlockSpec((HP, HP), lambda i: (0, 0)),
        ],
        out_specs=[pl.BlockSpec((BN, HP), lambda i: (i, 0)),
                   pl.BlockSpec((BN, HP), lambda i: (i, 0))],
        out_shape=[jax.ShapeDtypeStruct((NP, HP), jnp.float32)] * 2,
    )(h, WeA, beP, WeB)


def _readout_body(s0_ref, s1_ref, batch_ref, wr1_ref, br1_ref,
                  wr2_ref, br2_ref, o_ref):
    s = s0_ref[...] + s1_ref[...]
    gids = lax.broadcasted_iota(jnp.int32, (NG, NP), 0)
    onehot = (gids == batch_ref[...]).astype(jnp.float32)
    pooled = jnp.dot(onehot, s, preferred_element_type=jnp.float32,
                     precision=lax.Precision.HIGHEST)
    r1 = jax.nn.relu(jnp.dot(pooled[:, :DOUT], wr1_ref[...],
                             preferred_element_type=jnp.float32) + br1_ref[...])
    o_ref[...] = jnp.dot(r1, wr2_ref[...],
                         preferred_element_type=jnp.float32) + br2_ref[...]


def _readout(s0, s1, batch_r, Wr1, br1, Wr2, br2):
    return pl.pallas_call(
        _readout_body,
        in_specs=[
            pl.BlockSpec((NP, HP), lambda: (0, 0)),
            pl.BlockSpec((NP, HP), lambda: (0, 0)),
            pl.BlockSpec((1, NP), lambda: (0, 0)),
            pl.BlockSpec((DOUT, 64), lambda: (0, 0)),
            pl.BlockSpec((1, 64), lambda: (0, 0)),
            pl.BlockSpec((64, 1), lambda: (0, 0)),
            pl.BlockSpec((1, 1), lambda: (0, 0)),
        ],
        out_specs=pl.BlockSpec((NG, 1), lambda: (0, 0)),
        out_shape=jax.ShapeDtypeStruct((NG, 1), jnp.float32),
    )(s0, s1, batch_r, Wr1, br1, Wr2, br2)


SB = 2
TOTCH = EP // CH
CF, CS = 120, 40
IDXPAD = 16 * CF + 16 * CS + (CF - CS)
IDXPAD = ((IDXPAD + 7) // 8) * 8


def _sc_gather2(P1, P2, src_f, dst_f):
    @functools.partial(
        pl.kernel,
        mesh=_SC_MESH,
        out_type=(jax.ShapeDtypeStruct((EP, HP), jnp.float32),
                  jax.ShapeDtypeStruct((EP, HP), jnp.float32)),
        scratch_types=[
            pltpu.VMEM((CF, CH), jnp.int32),
            pltpu.VMEM((CF, CH), jnp.int32),
            pltpu.VMEM((SB * CH, HP), jnp.float32),
            pltpu.VMEM((SB * CH, HP), jnp.float32),
            pltpu.SemaphoreType.DMA,
            pltpu.SemaphoreType.DMA,
        ],
    )
    def k(p1_hbm, p2_hbm, src_hbm, dst_hbm, a_hbm, b_hbm,
          si_v, di_v, stag_a, stag_b, sema, semb):
        cid = lax.axis_index("c")
        sid = lax.axis_index("s")
        nsb = jnp.where(cid == 1, CF // SB, CS // SB)
        chunk0 = jnp.where(cid == 1, sid * CF, NS * CF + sid * CS)
        pltpu.sync_copy(src_hbm.at[pl.ds(chunk0, CF)], si_v)
        pltpu.sync_copy(dst_hbm.at[pl.ds(chunk0, CF)], di_v)

        def fire(table, idxv, k_sb, stag, sem):
            for j in range(SB):
                pltpu.async_copy(table.at[idxv.at[k_sb * SB + j]],
                                 stag.at[pl.ds(j * CH, CH)], sem)

        def drain(stag, sem):
            pltpu.make_async_copy(a_hbm.at[pl.ds(0, SB * CH)], stag, sem).wait()

        def out_slice(k_sb):
            return pl.ds((chunk0 + k_sb * SB) * CH, SB * CH)

        fire(p1_hbm, si_v, 0, stag_a, sema)

        @pl.loop(0, CF // SB)
        def _(k_sb):
            @pl.when(k_sb < nsb)
            def _():
                fire(p2_hbm, di_v, k_sb, stag_b, semb)
                drain(stag_a, sema)
                pltpu.sync_copy(stag_a, a_hbm.at[out_slice(k_sb)])

                @pl.when(k_sb + 1 < nsb)
                def _():
                    fire(p1_hbm, si_v, k_sb + 1, stag_a, sema)

                drain(stag_b, semb)
                pltpu.sync_copy(stag_b, b_hbm.at[out_slice(k_sb)])

    return k(P1, P2, src_f, dst_f)


def _sc_scatter(e, idx_r, zeros_np):
    @functools.partial(
        pl.kernel,
        mesh=_SC_MESH,
        out_type=jax.ShapeDtypeStruct((NC, NP, HP), jnp.float32),
        scratch_types=[
            pltpu.VMEM((NCHUNK, CH), jnp.int32),
            pltpu.VMEM((CH, HP), jnp.float32),
            pltpu.VMEM((CH, HP), jnp.float32),
            pltpu.VMEM_SHARED((NP, HP), jnp.float32),
            pltpu.SemaphoreType.DMA,
            pltpu.SemaphoreType.DMA,
        ],
    )
    def k(e_hbm, idx_hbm, z_hbm, out_hbm, idx_v, buf0, buf1, acc_sh, sem0, sem1):
        cid = lax.axis_index("c")
        sid = lax.axis_index("s")
        wid = cid * NS + sid
        base = wid * EPW
        rows = NP // NS
        pltpu.sync_copy(z_hbm.at[pl.ds(sid * rows, rows)],
                        acc_sh.at[pl.ds(sid * rows, rows)])
        plsc.subcore_barrier()
        pltpu.sync_copy(idx_hbm.at[wid], idx_v)

        bufs, sems = (buf0, buf1), (sem0, sem1)

        def lstart(ci, b):
            pltpu.async_copy(e_hbm.at[pl.ds(base + ci * CH, CH)], bufs[b], sems[b])

        lstart(0, 0)

        @pl.loop(0, NCHUNK, step=2)
        def _(c):
            for b in range(2):
                cc = c + b

                @pl.when(cc + 1 < NCHUNK)
                def _():
                    lstart(cc + 1, 1 - b)

                pltpu.make_async_copy(e_hbm.at[pl.ds(0, CH)],
                                      bufs[b], sems[b]).wait()
                pltpu.sync_copy(bufs[b], acc_sh.at[idx_v.at[cc]], add=True)

        plsc.subcore_barrier()

        @pl.when(sid == 0)
        def _():
            pltpu.sync_copy(acc_sh, out_hbm.at[cid])

    return k(e, idx_r, zeros_np)


def kernel(x, edge_attr, edge_index, batch, Wex, bex, Wee, bee, Wni, bni,
           Wei, bei, We, be, Wn, bn, Weo, beo, Wr1, br1, Wr2, br2):
    f32 = jnp.float32
    pad_w = lambda w: jnp.pad(w, ((0, HP - w.shape[0]), (0, HP - w.shape[1])))
    pad_c = lambda w: jnp.pad(w, ((0, 0), (0, HP - w.shape[1])))
    pad_b = lambda b: jnp.pad(b, (0, HP - b.shape[0])).reshape(1, HP)

    xp = jnp.pad(x, ((0, NP - N), (0, 0)))
    eap = jnp.pad(edge_attr, ((0, EP - E), (0, 0)))
    src_p = jnp.pad(edge_index[0], (0, EP - E), constant_values=N)
    dst_p = jnp.pad(edge_index[1], (0, EP - E), constant_values=N)
    flat = lambda v: jnp.pad(v.reshape(TOTCH, CH), ((0, IDXPAD - TOTCH), (0, 0)),
                             constant_values=N)
    src_f, dst_f = flat(src_p), flat(dst_p)
    src_r = src_p.reshape(NW, NCHUNK, CH)
    dst_r = dst_p.reshape(NW, NCHUNK, CH)
    batch_r = jnp.pad(batch, (0, NP - N), constant_values=NG).reshape(1, NP)

    Wni_p, bni_p = pad_c(Wni), pad_b(bni)
    Wei_p, bei_p = pad_c(Wei), pad_b(bei)
    WeA = [pad_w(We[l][:H]) for l in range(3)]
    WeB = [pad_w(We[l][H:2 * H]) for l in range(3)]
    W3 = [pad_w(We[l][2 * H:]) for l in range(3)]
    beP = [pad_b(be[l]) for l in range(3)]
    WnA = [pad_w(Wn[l][:H]) for l in range(2)]
    WnB = [pad_w(Wn[l][H:]) for l in range(2)]
    bn_p = [pad_b(bn[l]) for l in range(2)]
    Weo_aug = jnp.zeros((HP, HP), f32)
    Weo_aug = Weo_aug.at[:H, :DOUT].set(Weo).at[H, :DOUT].set(beo)
    zeros_np = jnp.zeros((NP, HP), f32)

    h = _node_init(xp, Wex, bex.reshape(1, DX), Wni_p, bni_p)
    e = _edge_init(eap, Wee, bee.reshape(1, DX), Wei_p, bei_p)
    P1, P2 = _pp(h, WeA[0], beP[0], WeB[0])

    for l in range(3):
        A, B = _sc_gather2(P1, P2, src_f, dst_f)
        if l < 2:
            e = _edge_update(e, A, B, W3[l])
            parts = _sc_scatter(e, dst_r, zeros_np)
            h, P1, P2 = _node_update_pp(h, parts[0], parts[1], WnA[l], WnB[l],
                                        bn_p[l], WeA[l + 1], beP[l + 1], WeB[l + 1])
        else:
            e_out = _edge_out(e, A, B, W3[l], Weo_aug)
            parts = _sc_scatter(e_out, src_r, zeros_np)

    return _readout(parts[0], parts[1], batch_r,
                    Wr1, br1.reshape(1, 64), Wr2, br2.reshape(1, 1))
